# baseline (speedup 1.0000x reference)
import functools

import jax
import jax.numpy as jnp
from jax import lax
from jax.experimental import pallas as pl
from jax.experimental.pallas import tpu as pltpu
from jax.experimental.pallas import tpu_sc as plsc

NUM_FEATURES = 26
VOCAB = 100000
EMBED = 32
BATCH = 16384

NC = 2
NS = 16
NW = NC * NS
LANES = 16

VCHUNK = 12800
N_VCHUNKS = 8
VTAIL = VOCAB - 7 * VCHUNK
SUB = VCHUNK // 4
SUB_T = VTAIL // 4
ROWS_PER_F = VOCAB * EMBED // 128
N_ROWS128 = NUM_FEATURES * ROWS_PER_F

B_W = BATCH // NW
IDX_SLICE = 128
GATHERS = B_W // IDX_SLICE

BCHUNK = 2048
N_BCHUNKS = BATCH // BCHUNK


def _rows_body(tt_ref, tail_ref, t128_ref, inbuf, outbuf, inbuf_t, outbuf_t, sem_i, sem_o):
    f = pl.program_id(0)
    c = pl.program_id(1)

    @pl.when(c < 7)
    def _():
        v0 = pl.multiple_of(c * VCHUNK, 128)
        cp_i = pltpu.make_async_copy(tt_ref.at[f, :, pl.ds(v0, VCHUNK)], inbuf, sem_i)
        cp_i.start()
        cp_i.wait()
        for u in range(4):
            outbuf[:, pl.ds(EMBED * u, EMBED)] = inbuf[:, pl.ds(SUB * u, SUB)].T
        row0 = f * ROWS_PER_F + c * SUB
        cp_o = pltpu.make_async_copy(outbuf, t128_ref.at[pl.ds(row0, SUB), :], sem_o)
        cp_o.start()
        cp_o.wait()

    @pl.when(c == 7)
    def _():
        cp_i = pltpu.make_async_copy(tail_ref.at[f], inbuf_t, sem_i)
        cp_i.start()
        cp_i.wait()
        whole = inbuf_t[...]
        for u in range(4):
            outbuf_t[:, pl.ds(EMBED * u, EMBED)] = lax.slice(
                whole, (0, SUB_T * u), (EMBED, SUB_T * (u + 1))
            ).T
        row0 = f * ROWS_PER_F + 7 * SUB
        cp_o = pltpu.make_async_copy(
            outbuf_t, t128_ref.at[pl.ds(row0, SUB_T), :], sem_o
        )
        cp_o.start()
        cp_o.wait()


def _out_body(i_ref, o_ref):
    for j in range(4):
        o_ref[0, :, pl.ds(512 * j, 512)] = i_ref[0, :, pl.ds(EMBED * j, EMBED)].T


def _gather_body(xt, tab, out, idx_v, rows_v, gsem):
    wid = lax.axis_index("s") * NC + lax.axis_index("c")
    b0 = wid * B_W
    chunk = wid // 4
    jj = wid - chunk * 4

    one = jnp.full((LANES,), 1, jnp.int32)
    zero = jnp.full((LANES,), 0, jnp.int32)

    def feature_body(f, carry):
        pltpu.sync_copy(xt.at[f, pl.ds(b0, B_W)], idx_v)
        for k in range(B_W // LANES):
            sl = pl.ds(k * LANES, LANES)
            v = idx_v[sl]
            c7 = zero
            for t in range(1, N_VCHUNKS):
                c7 = c7 + jnp.where(v >= t * VCHUNK, one, zero)
            w = v - c7 * VCHUNK
            sub = jnp.where(c7 >= 7, jnp.full((LANES,), SUB_T, jnp.int32),
                            jnp.full((LANES,), SUB, jnp.int32))
            u3 = (
                jnp.where(w >= sub, one, zero)
                + jnp.where(w >= sub + sub, one, zero)
                + jnp.where(w >= sub * 3, one, zero)
            )
            idx_v[sl] = c7 * VCHUNK + (w - u3 * sub) * 4 + u3
        copies = []
        for k in range(GATHERS):
            cp = pltpu.async_copy(
                tab.at[f].at[idx_v.at[pl.ds(k * IDX_SLICE, IDX_SLICE)]],
                rows_v.at[pl.ds(k * IDX_SLICE, IDX_SLICE)],
                gsem,
            )
            copies.append(cp)
        for cp in copies:
            cp.wait()
        pltpu.sync_copy(rows_v, out.at[f, chunk, :, jj, :])
        return carry

    lax.fori_loop(0, NUM_FEATURES, feature_body, 0)


def kernel(x, tables):
    xt = x.T
    tt = tables.transpose(0, 2, 1)

    t128 = pl.pallas_call(
        _rows_body,
        out_shape=jax.ShapeDtypeStruct((N_ROWS128, 128), jnp.float32),
        grid=(NUM_FEATURES, N_VCHUNKS),
        in_specs=[
            pl.BlockSpec(memory_space=pl.ANY),
            pl.BlockSpec(memory_space=pl.ANY),
        ],
        out_specs=pl.BlockSpec(memory_space=pl.ANY),
        scratch_shapes=[
            pltpu.VMEM((EMBED, VCHUNK), jnp.float32),
            pltpu.VMEM((SUB, 128), jnp.float32),
            pltpu.VMEM((EMBED, VTAIL), jnp.float32),
            pltpu.VMEM((SUB_T, 128), jnp.float32),
            pltpu.SemaphoreType.DMA,
            pltpu.SemaphoreType.DMA,
        ],
    )(tt, lax.slice(tt, (0, 0, 7 * VCHUNK), (NUM_FEATURES, EMBED, VOCAB)))

    gat = functools.partial(
        pl.kernel,
        out_type=jax.ShapeDtypeStruct(
            (NUM_FEATURES, N_BCHUNKS, 512, 4, EMBED), jnp.float32
        ),
        mesh=plsc.VectorSubcoreMesh(core_axis_name="c", subcore_axis_name="s"),
        compiler_params=pltpu.CompilerParams(use_tc_tiling_on_sc=False),
        scratch_types=[
            pltpu.VMEM((B_W,), jnp.int32),
            pltpu.VMEM((B_W, EMBED), jnp.float32),
            pltpu.SemaphoreType.DMA,
        ],
    )(_gather_body)

    tab3 = t128.reshape(NUM_FEATURES, VOCAB, EMBED)
    out_t = gat(xt, tab3)

    o3 = pl.pallas_call(
        _out_body,
        out_shape=jax.ShapeDtypeStruct((NUM_FEATURES, EMBED, BATCH), jnp.float32),
        grid=(NUM_FEATURES, N_BCHUNKS),
        in_specs=[pl.BlockSpec((1, 512, 128), lambda f, c: (f, c, 0))],
        out_specs=pl.BlockSpec((1, EMBED, BCHUNK), lambda f, c: (f, 0, c)),
    )(out_t.reshape(NUM_FEATURES, BATCH // 4, 128))

    return o3.transpose(2, 0, 1)


# speedup vs baseline: 1.6470x; 1.6470x over previous
import functools

import jax
import jax.numpy as jnp
from jax import lax
from jax.experimental import pallas as pl
from jax.experimental.pallas import tpu as pltpu
from jax.experimental.pallas import tpu_sc as plsc

NUM_FEATURES = 26
VOCAB = 100000
EMBED = 32
BATCH = 16384

NC = 2
NS = 16
NW = NC * NS
LANES = 16

VCHUNK = 12800
N_VCHUNKS = 8
SUB = VCHUNK // 4
ROWS_PER_F = N_VCHUNKS * SUB
VPAD_F = ROWS_PER_F * 4

B_W = BATCH // NW
IDX_SLICE = 128
GATHERS = B_W // IDX_SLICE

BCHUNK = 2048
N_BCHUNKS = BATCH // BCHUNK


def _eye(n):
    r = lax.broadcasted_iota(jnp.int32, (n, n), 0)
    c = lax.broadcasted_iota(jnp.int32, (n, n), 1)
    return jnp.where(r == c, jnp.float32(1), jnp.float32(0))


def _rows_body(i_ref, o_ref):
    ident = _eye(EMBED)
    for u in range(4):
        o_ref[:, pl.ds(EMBED * u, EMBED)] = lax.dot_general(
            i_ref[0, :, pl.ds(SUB * u, SUB)],
            ident,
            (((0,), (0,)), ((), ())),
            preferred_element_type=jnp.float32,
        )


def _out_body(i_ref, o_ref):
    ident = _eye(EMBED)
    for j in range(4):
        o_ref[0, :, pl.ds(512 * j, 512)] = lax.dot_general(
            ident,
            i_ref[0, :, pl.ds(EMBED * j, EMBED)],
            (((1,), (1,)), ((), ())),
            preferred_element_type=jnp.float32,
        )


def _gather_body(xt, tab, out, idx_v, rows_v, gsem):
    wid = lax.axis_index("s") * NC + lax.axis_index("c")
    b0 = wid * B_W
    chunk = wid // 4
    jj = wid - chunk * 4

    one = jnp.full((LANES,), 1, jnp.int32)
    zero = jnp.full((LANES,), 0, jnp.int32)

    def feature_body(f, carry):
        pltpu.sync_copy(xt.at[f, pl.ds(b0, B_W)], idx_v)
        for k in range(B_W // LANES):
            sl = pl.ds(k * LANES, LANES)
            v = idx_v[sl]
            c7 = zero
            for t in range(1, N_VCHUNKS):
                c7 = c7 + jnp.where(v >= t * VCHUNK, one, zero)
            w = v - c7 * VCHUNK
            u3 = (
                jnp.where(w >= SUB, one, zero)
                + jnp.where(w >= 2 * SUB, one, zero)
                + jnp.where(w >= 3 * SUB, one, zero)
            )
            idx_v[sl] = c7 * VCHUNK + (w - u3 * SUB) * 4 + u3
        copies = []
        for k in range(GATHERS):
            cp = pltpu.async_copy(
                tab.at[f].at[idx_v.at[pl.ds(k * IDX_SLICE, IDX_SLICE)]],
                rows_v.at[pl.ds(k * IDX_SLICE, IDX_SLICE)],
                gsem,
            )
            copies.append(cp)
        for cp in copies:
            cp.wait()
        pltpu.sync_copy(rows_v, out.at[f, chunk, :, jj, :])
        return carry

    lax.fori_loop(0, NUM_FEATURES, feature_body, 0)


def kernel(x, tables):
    xt = x.T
    tt = tables.transpose(0, 2, 1)

    t128 = pl.pallas_call(
        _rows_body,
        out_shape=jax.ShapeDtypeStruct((NUM_FEATURES * ROWS_PER_F, 128), jnp.float32),
        grid=(NUM_FEATURES, N_VCHUNKS),
        in_specs=[pl.BlockSpec((1, EMBED, VCHUNK), lambda f, c: (f, 0, c))],
        out_specs=pl.BlockSpec((SUB, 128), lambda f, c: (f * N_VCHUNKS + c, 0)),
    )(tt)

    gat = functools.partial(
        pl.kernel,
        out_type=jax.ShapeDtypeStruct(
            (NUM_FEATURES, N_BCHUNKS, 512, 4, EMBED), jnp.float32
        ),
        mesh=plsc.VectorSubcoreMesh(core_axis_name="c", subcore_axis_name="s"),
        compiler_params=pltpu.CompilerParams(use_tc_tiling_on_sc=False),
        scratch_types=[
            pltpu.VMEM((B_W,), jnp.int32),
            pltpu.VMEM((B_W, EMBED), jnp.float32),
            pltpu.SemaphoreType.DMA,
        ],
    )(_gather_body)

    tab3 = t128.reshape(NUM_FEATURES, VPAD_F, EMBED)
    out_t = gat(xt, tab3)

    o3 = pl.pallas_call(
        _out_body,
        out_shape=jax.ShapeDtypeStruct((NUM_FEATURES, EMBED, BATCH), jnp.float32),
        grid=(NUM_FEATURES, N_BCHUNKS),
        in_specs=[pl.BlockSpec((1, 512, 128), lambda f, c: (f, c, 0))],
        out_specs=pl.BlockSpec((1, EMBED, BCHUNK), lambda f, c: (f, 0, c)),
    )(out_t.reshape(NUM_FEATURES, BATCH // 4, 128))

    return o3.transpose(2, 0, 1)


# speedup vs baseline: 1.6700x; 1.0140x over previous
import functools

import jax
import jax.numpy as jnp
from jax import lax
from jax.experimental import pallas as pl
from jax.experimental.pallas import tpu as pltpu
from jax.experimental.pallas import tpu_sc as plsc

NUM_FEATURES = 26
VOCAB = 100000
EMBED = 32
BATCH = 16384

NC = 2
NS = 16
NW = NC * NS
LANES = 16

VCHUNK = 25600
N_VCHUNKS = 4
SUB = VCHUNK // 4
ROWS_PER_F = N_VCHUNKS * SUB
VPAD_F = ROWS_PER_F * 4

B_W = BATCH // NW
IDX_SLICE = 128
GATHERS = B_W // IDX_SLICE

BCHUNK = 2048
N_BCHUNKS = BATCH // BCHUNK


def _eye(n):
    r = lax.broadcasted_iota(jnp.int32, (n, n), 0)
    c = lax.broadcasted_iota(jnp.int32, (n, n), 1)
    return jnp.where(r == c, jnp.float32(1), jnp.float32(0))


def _rows_body(i_ref, o_ref):
    ident = _eye(EMBED)
    for u in range(4):
        o_ref[:, pl.ds(EMBED * u, EMBED)] = lax.dot_general(
            i_ref[0, :, pl.ds(SUB * u, SUB)],
            ident,
            (((0,), (0,)), ((), ())),
            preferred_element_type=jnp.float32,
        )


def _out_body(i_ref, o_ref):
    ident = _eye(EMBED)
    for j in range(4):
        o_ref[0, :, pl.ds(512 * j, 512)] = lax.dot_general(
            ident,
            i_ref[0, :, pl.ds(EMBED * j, EMBED)],
            (((1,), (1,)), ((), ())),
            preferred_element_type=jnp.float32,
        )


def _gather_body(xt, tab, out, idx_v, rows_v, gsem):
    wid = lax.axis_index("s") * NC + lax.axis_index("c")
    b0 = wid * B_W
    chunk = wid // 4
    jj = wid - chunk * 4

    one = jnp.full((LANES,), 1, jnp.int32)
    zero = jnp.full((LANES,), 0, jnp.int32)

    def feature_body(f, carry):
        pltpu.sync_copy(xt.at[f, pl.ds(b0, B_W)], idx_v)
        for k in range(B_W // LANES):
            sl = pl.ds(k * LANES, LANES)
            v = idx_v[sl]
            c7 = zero
            for t in range(1, N_VCHUNKS):
                c7 = c7 + jnp.where(v >= t * VCHUNK, one, zero)
            w = v - c7 * VCHUNK
            u3 = (
                jnp.where(w >= SUB, one, zero)
                + jnp.where(w >= 2 * SUB, one, zero)
                + jnp.where(w >= 3 * SUB, one, zero)
            )
            idx_v[sl] = c7 * VCHUNK + (w - u3 * SUB) * 4 + u3
        copies = []
        for k in range(GATHERS):
            cp = pltpu.async_copy(
                tab.at[f].at[idx_v.at[pl.ds(k * IDX_SLICE, IDX_SLICE)]],
                rows_v.at[pl.ds(k * IDX_SLICE, IDX_SLICE)],
                gsem,
            )
            copies.append(cp)
        for cp in copies:
            cp.wait()
        pltpu.sync_copy(rows_v, out.at[f, chunk, :, jj, :])
        return carry

    lax.fori_loop(0, NUM_FEATURES, feature_body, 0)


def kernel(x, tables):
    xt = x.T
    tt = tables.transpose(0, 2, 1)

    t128 = pl.pallas_call(
        _rows_body,
        out_shape=jax.ShapeDtypeStruct((NUM_FEATURES * ROWS_PER_F, 128), jnp.float32),
        grid=(NUM_FEATURES, N_VCHUNKS),
        in_specs=[pl.BlockSpec((1, EMBED, VCHUNK), lambda f, c: (f, 0, c))],
        out_specs=pl.BlockSpec((SUB, 128), lambda f, c: (f * N_VCHUNKS + c, 0)),
    )(tt)

    gat = functools.partial(
        pl.kernel,
        out_type=jax.ShapeDtypeStruct(
            (NUM_FEATURES, N_BCHUNKS, 512, 4, EMBED), jnp.float32
        ),
        mesh=plsc.VectorSubcoreMesh(core_axis_name="c", subcore_axis_name="s"),
        compiler_params=pltpu.CompilerParams(use_tc_tiling_on_sc=False),
        scratch_types=[
            pltpu.VMEM((B_W,), jnp.int32),
            pltpu.VMEM((B_W, EMBED), jnp.float32),
            pltpu.SemaphoreType.DMA,
        ],
    )(_gather_body)

    tab3 = t128.reshape(NUM_FEATURES, VPAD_F, EMBED)
    out_t = gat(xt, tab3)

    o3 = pl.pallas_call(
        _out_body,
        out_shape=jax.ShapeDtypeStruct((NUM_FEATURES, EMBED, BATCH), jnp.float32),
        grid=(NUM_FEATURES, N_BCHUNKS),
        in_specs=[pl.BlockSpec((1, 512, 128), lambda f, c: (f, c, 0))],
        out_specs=pl.BlockSpec((1, EMBED, BCHUNK), lambda f, c: (f, 0, c)),
    )(out_t.reshape(NUM_FEATURES, BATCH // 4, 128))

    return o3.transpose(2, 0, 1)
